# baseline (speedup 1.0000x reference)
import functools

import jax
import jax.numpy as jnp
from jax import lax
from jax.experimental import pallas as pl
from jax.experimental.pallas import tpu as pltpu
from jax.experimental.pallas import tpu_sc as plsc

B = 16384
D = 16
V = 1000000
V8 = V * D // 128

_NC, _NS = 2, 16
_NW = _NC * _NS
_BPW = B // _NW
_NG = _BPW // 16


def _extract(idx_v, rows_v, out_v, f0):
  lanes = lax.iota(jnp.int32, 16)

  def body(g, _):
    j0 = g * 16
    idx16 = idx_v[pl.ds(j0, 16)]
    off = (idx16 & 7) << 4
    rows = j0 + lanes
    for f in range(D):
      vals = plsc.load_gather(rows_v, [rows, off + f])
      out_v[f0 + f, pl.ds(j0, 16)] = vals
    return 0

  lax.fori_loop(0, _NG, body, 0, unroll=False)


def _sc_gather_body(x0_hbm, x1_hbm, x2_hbm, x3_hbm,
                    t0_hbm, t1_hbm, t2_hbm, t3_hbm,
                    gu_hbm, gi_hbm, x_hbm,
                    i0_v, i1_v, i2_v, i3_v,
                    i8_v, rows_v, gu_v, gi_v, x_v,
                    sem):
  wid = lax.axis_index("s") * _NC + lax.axis_index("c")
  base = wid * _BPW

  def one_table(t_hbm, idx_hbm, idx_v, out_v, f0):
    pltpu.sync_copy(idx_hbm.at[pl.ds(base, _BPW)], idx_v)

    def i8body(i, _):
      i8_v[pl.ds(i * 16, 16)] = idx_v[pl.ds(i * 16, 16)] >> 3
      return 0

    lax.fori_loop(0, _BPW // 16, i8body, 0, unroll=False)
    pltpu.async_copy(t_hbm.at[i8_v], rows_v, sem).wait()
    _extract(idx_v, rows_v, out_v, f0)

  for t_hbm, idx_hbm, idx_v, out_v, f0 in (
      (t0_hbm, x0_hbm, i0_v, gu_v, 0),
      (t1_hbm, x1_hbm, i1_v, gi_v, 0),
      (t2_hbm, x2_hbm, i2_v, x_v, 0),
      (t3_hbm, x3_hbm, i3_v, x_v, D),
  ):
    one_table(t_hbm, idx_hbm, idx_v, out_v, f0)

  pltpu.sync_copy(gu_v, gu_hbm.at[pl.ds(0, D), pl.ds(base, _BPW)])
  pltpu.sync_copy(gi_v, gi_hbm.at[pl.ds(0, D), pl.ds(base, _BPW)])
  pltpu.sync_copy(x_v, x_hbm.at[pl.ds(0, 2 * D), pl.ds(base, _BPW)])


_sc_gather = functools.partial(
    pl.kernel,
    mesh=plsc.VectorSubcoreMesh(core_axis_name="c", subcore_axis_name="s"),
    compiler_params=pltpu.CompilerParams(needs_layout_passes=False),
    out_type=[
        jax.ShapeDtypeStruct((D, B), jnp.float32),
        jax.ShapeDtypeStruct((D, B), jnp.float32),
        jax.ShapeDtypeStruct((2 * D, B), jnp.float32),
    ],
    scratch_types=(
        [pltpu.VMEM((_BPW,), jnp.int32)] * 5
        + [pltpu.VMEM((_BPW, 128), jnp.float32)]
        + [pltpu.VMEM((D, _BPW), jnp.float32)] * 2
        + [pltpu.VMEM((2 * D, _BPW), jnp.float32)]
        + [pltpu.SemaphoreType.DMA]
    ),
)(_sc_gather_body)


_COLS = 1024


def _tc_head_body(gu, gi, xt, w1t, b1c, w2t, b2c, gwf, w3f, cb, out):
  f32 = jnp.float32
  g = gu[...] * gi[...]
  h1 = jnp.maximum(
      jnp.dot(w1t[...], xt[...], preferred_element_type=f32) + b1c[...], 0.0)
  h2 = jnp.maximum(
      jnp.dot(w2t[...], h1, preferred_element_type=f32) + b2c[...], 0.0)
  z = jnp.dot(gwf[...], g, preferred_element_type=f32)
  z = z + jnp.dot(w3f[...], h2, preferred_element_type=f32) + cb[...]
  out[...] = jax.nn.sigmoid(z)


def _tc_head(gu, gi, xt, w1t, b1c, w2t, b2c, gwf, w3f, cb):
  n_steps = B // _COLS
  col = lambda r: pl.BlockSpec((r, _COLS), lambda i: (0, i))
  full = lambda a: pl.BlockSpec(a.shape, lambda i: (0,) * a.ndim)
  return pl.pallas_call(
      _tc_head_body,
      grid=(n_steps,),
      in_specs=[col(D), col(D), col(2 * D), full(w1t), full(b1c),
                full(w2t), full(b2c), full(gwf), full(w3f), full(cb)],
      out_specs=pl.BlockSpec((1, _COLS), lambda i: (0, i)),
      out_shape=jax.ShapeDtypeStruct((1, B), jnp.float32),
  )(gu, gi, xt, w1t, b1c, w2t, b2c, gwf, w3f, cb)


def kernel(x0, x1, x2, x3, gmf_user_emb, gmf_item_emb, gmf_w, gmf_b,
           mlp_user_emb, mlp_item_emb, w1, b1, w2, b2, w3, b3, cls_w, cls_b):
  i0 = x0.reshape(B).astype(jnp.int32)
  i1 = x1.reshape(B).astype(jnp.int32)
  i2 = x2.reshape(B).astype(jnp.int32)
  i3 = x3.reshape(B).astype(jnp.int32)

  t0 = gmf_user_emb.reshape(V8, 128)
  t1 = gmf_item_emb.reshape(V8, 128)
  t2 = mlp_user_emb.reshape(V8, 128)
  t3 = mlp_item_emb.reshape(V8, 128)

  gu, gi, xt = _sc_gather(i0, i1, i2, i3, t0, t1, t2, t3)

  cw0 = cls_w[0, 0]
  cw1 = cls_w[1, 0]
  w1t = w1.T
  b1c = b1.reshape(2 * D, 1)
  w2t = w2.T
  b2c = b2.reshape(D, 1)
  gwf = (gmf_w * cw0).T
  w3f = (w3 * cw1).T
  cb = jnp.full((1, 1), gmf_b[0] * cw0 + b3[0] * cw1 + cls_b[0],
                dtype=jnp.float32)

  out = _tc_head(gu, gi, xt, w1t, b1c, w2t, b2c, gwf, w3f, cb)
  return out.reshape(B, 1)
